# baseline (speedup 1.0000x reference)
import functools

import jax
import jax.numpy as jnp
from jax import lax
from jax.experimental import pallas as pl
from jax.experimental.pallas import tpu as pltpu
from jax.experimental.pallas import tpu_sc as plsc

N = 10000
E = 160000
D_IN = 256
D_H = 512
G = 64
N_CLS = 6

NSC = 2
NSUB = 16
CH = 128
NPASS = 4
ROWS = 10112
RPS = ROWS // NSUB
KD = 40
E_DEG = 32 * KD * 128
KCA = 40
NCHA = 255
NSLOT = 5
DPRE = 3
E_AGG = NSUB * NCHA * KCA
RB = 1000
NRB = N // RB

_mesh = plsc.VectorSubcoreMesh(core_axis_name="c", subcore_axis_name="s",
                               num_cores=NSC, num_subcores=NSUB)


@functools.partial(
    pl.kernel,
    out_type=jax.ShapeDtypeStruct((NSC, ROWS, CH), jnp.float32),
    scratch_types=[
        pltpu.VMEM((128,), jnp.int32),
        pltpu.VMEM((128, CH), jnp.float32),
        pltpu.VMEM_SHARED((ROWS, CH), jnp.float32),
    ],
    mesh=_mesh,
)
def _deg_sc(dst_hbm, ones_hbm, zero_hbm, out_hbm, idx_v, ones_v, acc_sh):
    c = lax.axis_index("c")
    s = lax.axis_index("s")
    w = s * NSC + c
    pltpu.sync_copy(ones_hbm, ones_v)
    pltpu.sync_copy(zero_hbm, acc_sh.at[pl.ds(s * RPS, RPS)])
    plsc.subcore_barrier()

    def body(j, carry):
        pltpu.sync_copy(dst_hbm.at[w, j], idx_v)
        pltpu.sync_copy(ones_v, acc_sh.at[idx_v], add=True)
        return carry

    lax.fori_loop(0, KD, body, 0)
    plsc.subcore_barrier()
    pltpu.sync_copy(acc_sh.at[pl.ds(s * RPS, RPS)],
                    out_hbm.at[c, pl.ds(s * RPS, RPS)])


@functools.partial(
    pl.kernel,
    out_type=jax.ShapeDtypeStruct((NPASS, ROWS, CH), jnp.float32),
    scratch_types=(
        [pltpu.VMEM((NCHA * KCA,), jnp.int32),
         pltpu.VMEM((NCHA * KCA,), jnp.int32)]
        + [pltpu.VMEM((KCA, CH), jnp.float32) for _ in range(NSLOT)]
        + [pltpu.VMEM_SHARED((ROWS, CH), jnp.float32)]
        + [pltpu.SemaphoreType.DMA for _ in range(2 * NSLOT)]
    ),
    mesh=_mesh,
)
def _agg_sc(table_hbm, src_hbm, dst_hbm, zero_hbm, out_hbm,
            srcv, dstv, *rest):
    sts = rest[:NSLOT]
    acc_sh = rest[NSLOT]
    gsem = rest[NSLOT + 1:2 * NSLOT + 1]
    ssem = rest[2 * NSLOT + 1:]
    c = lax.axis_index("c")
    s = lax.axis_index("s")
    pltpu.sync_copy(dst_hbm.at[s], dstv)

    def gather(j, b):
        pltpu.async_copy(table_hbm.at[srcv.at[pl.ds(j * KCA, KCA)]],
                         sts[b], gsem[b])

    def gwait(b):
        pltpu.make_async_copy(table_hbm.at[srcv.at[pl.ds(0, KCA)]],
                              sts[b], gsem[b]).wait()

    def scatter(j, b):
        pltpu.async_copy(sts[b], acc_sh.at[dstv.at[pl.ds(j * KCA, KCA)]],
                         ssem[b], add=True)

    def swait(b):
        pltpu.make_async_copy(sts[b], acc_sh.at[dstv.at[pl.ds(0, KCA)]],
                              ssem[b]).wait()

    for k in range(2):
        p = c + 2 * k
        pltpu.sync_copy(src_hbm.at[p, s], srcv)
        pltpu.sync_copy(zero_hbm, acc_sh.at[pl.ds(s * RPS, RPS)])
        plsc.subcore_barrier()

        for b in range(DPRE):
            gather(b, b)
        gather(3, 3)
        gwait(0)
        scatter(0, 0)
        gather(4, 4)
        gwait(1)
        scatter(1, 1)

        def body(tt, cc):
            for v in range(NSLOT):
                j = 2 + NSLOT * tt + v
                swait(v)
                gather(j + DPRE, v)
                gwait((2 + v) % NSLOT)
                scatter(j, (2 + v) % NSLOT)
            return cc

        lax.fori_loop(0, (NCHA - DPRE - 2) // NSLOT, body, 0)
        for j in range(NCHA - DPRE, NCHA):
            gwait(j % NSLOT)
            scatter(j, j % NSLOT)
        for b in range(NSLOT):
            swait(b)
        plsc.subcore_barrier()
        pltpu.sync_copy(acc_sh.at[pl.ds(s * RPS, RPS)],
                        out_hbm.at[p, pl.ds(s * RPS, RPS)])
        if k == 0:
            plsc.subcore_barrier()


def _p0_body(x_ref, w_ref, deg_ref, hp_ref, dinv_ref):
    deg = deg_ref[0, :, 0:1] + deg_ref[1, :, 0:1] + 1.0
    dinv = lax.rsqrt(deg)
    h = jnp.dot(x_ref[...], w_ref[...], preferred_element_type=jnp.float32)
    hp = h * dinv
    dinv_ref[...] = dinv
    for ci in range(NPASS):
        hp_ref[ci] = hp[:, ci * CH:(ci + 1) * CH]


def _p12_body(agg_ref, hp_ref, dinv_ref, b_ref, w_ref, out_ref):
    dinv = dinv_ref[...]
    parts = []
    for ci in range(NPASS):
        t = dinv * (agg_ref[ci] + hp_ref[ci]) + b_ref[0:1, ci * CH:(ci + 1) * CH]
        parts.append(jnp.maximum(t, 0.0))
    t_full = jnp.concatenate(parts, axis=1)
    h = jnp.dot(t_full, w_ref[...], preferred_element_type=jnp.float32)
    hp = h * dinv
    for ci in range(NPASS):
        out_ref[ci] = hp[:, ci * CH:(ci + 1) * CH]


def _p3_body(agg_ref, hp_ref, dinv_ref, b_ref, batch_ref, wl_ref, bl_ref,
             hg_ref, lsm_ref, acc_ref, cnt_ref):
    i = pl.program_id(0)
    dinv = dinv_ref[...]
    parts = []
    for ci in range(NPASS):
        parts.append(dinv * (agg_ref[ci] + hp_ref[ci])
                     + b_ref[0:1, ci * CH:(ci + 1) * CH])
    h = jnp.concatenate(parts, axis=1)
    rows = lax.broadcasted_iota(jnp.int32, (G, RB), 0)
    oh = jnp.where(rows == batch_ref[0], 1.0, 0.0)
    pooled = jnp.dot(oh, h, preferred_element_type=jnp.float32)
    cnt = jnp.sum(oh, axis=1, keepdims=True)

    @pl.when(i == 0)
    def _init():
        acc_ref[...] = jnp.zeros((G, D_H), jnp.float32)
        cnt_ref[...] = jnp.zeros((G, 128), jnp.float32)

    acc_ref[...] += pooled
    cnt_ref[...] += jnp.broadcast_to(cnt, (G, 128))

    @pl.when(i == NRB - 1)
    def _fin():
        counts = cnt_ref[:, 0:1]
        hg = acc_ref[...] / jnp.maximum(counts, 1.0)
        hg_ref[...] = hg
        logits = jnp.dot(hg, wl_ref[...],
                         preferred_element_type=jnp.float32) + bl_ref[...]
        m = jnp.max(logits, axis=1, keepdims=True)
        ex = jnp.exp(logits - m)
        lsm_ref[...] = (logits - m) - jnp.log(jnp.sum(ex, axis=1, keepdims=True))


_f32 = jnp.float32

_p0 = pl.pallas_call(
    _p0_body,
    grid=(NRB,),
    in_specs=[
        pl.BlockSpec((RB, D_IN), lambda i: (i, 0)),
        pl.BlockSpec((D_IN, D_H), lambda i: (0, 0)),
        pl.BlockSpec((NSC, RB, CH), lambda i: (0, i, 0)),
    ],
    out_specs=[
        pl.BlockSpec((NPASS, RB, CH), lambda i: (0, i, 0)),
        pl.BlockSpec((RB, 1), lambda i: (i, 0)),
    ],
    out_shape=[
        jax.ShapeDtypeStruct((NPASS, N, CH), _f32),
        jax.ShapeDtypeStruct((N, 1), _f32),
    ],
)

_p12 = pl.pallas_call(
    _p12_body,
    grid=(NRB,),
    in_specs=[
        pl.BlockSpec((NPASS, RB, CH), lambda i: (0, i, 0)),
        pl.BlockSpec((NPASS, RB, CH), lambda i: (0, i, 0)),
        pl.BlockSpec((RB, 1), lambda i: (i, 0)),
        pl.BlockSpec((1, D_H), lambda i: (0, 0)),
        pl.BlockSpec((D_H, D_H), lambda i: (0, 0)),
    ],
    out_specs=[pl.BlockSpec((NPASS, RB, CH), lambda i: (0, i, 0))],
    out_shape=[jax.ShapeDtypeStruct((NPASS, N, CH), _f32)],
)

_p3 = pl.pallas_call(
    _p3_body,
    grid=(NRB,),
    in_specs=[
        pl.BlockSpec((NPASS, RB, CH), lambda i: (0, i, 0)),
        pl.BlockSpec((NPASS, RB, CH), lambda i: (0, i, 0)),
        pl.BlockSpec((RB, 1), lambda i: (i, 0)),
        pl.BlockSpec((1, D_H), lambda i: (0, 0)),
        pl.BlockSpec((1, 1, RB), lambda i: (i, 0, 0)),
        pl.BlockSpec((D_H, N_CLS), lambda i: (0, 0)),
        pl.BlockSpec((1, N_CLS), lambda i: (0, 0)),
    ],
    out_specs=[
        pl.BlockSpec((G, D_H), lambda i: (0, 0)),
        pl.BlockSpec((G, N_CLS), lambda i: (0, 0)),
    ],
    out_shape=[
        jax.ShapeDtypeStruct((G, D_H), _f32),
        jax.ShapeDtypeStruct((G, N_CLS), _f32),
    ],
    scratch_shapes=[
        pltpu.VMEM((G, D_H), _f32),
        pltpu.VMEM((G, 128), _f32),
    ],
)


def kernel(x, edge_index, batch, W1, b1, W2, b2, W3, b3, Wl, bl):
    src0 = edge_index[0]
    dst0 = edge_index[1]

    pad_d = jnp.full((E_DEG - E,), N, jnp.int32)
    dst_deg = jnp.concatenate([dst0, pad_d]).reshape(32, KD, 128)
    src_pad = jnp.concatenate([src0, jnp.zeros((E_AGG - E,), jnp.int32)])
    src_agg = (src_pad[None, :] + (jnp.arange(NPASS, dtype=jnp.int32) * N)[:, None]
               ).reshape(NPASS, NSUB, NCHA * KCA)
    dst_agg = jnp.concatenate(
        [dst0, jnp.full((E_AGG - E,), N, jnp.int32)]).reshape(NSUB, NCHA * KCA)

    ones128 = jnp.ones((128, CH), _f32)
    zero_agg = jnp.zeros((RPS, CH), _f32)

    degp = _deg_sc(dst_deg, ones128, zero_agg)

    h1p, dinv = _p0(x, W1, degp)
    agg1 = _agg_sc(h1p.reshape(NPASS * N, CH), src_agg, dst_agg, zero_agg)
    h2p, = _p12(agg1, h1p, dinv, b1.reshape(1, D_H), W2)
    agg2 = _agg_sc(h2p.reshape(NPASS * N, CH), src_agg, dst_agg, zero_agg)
    h3p, = _p12(agg2, h2p, dinv, b2.reshape(1, D_H), W3)
    agg3 = _agg_sc(h3p.reshape(NPASS * N, CH), src_agg, dst_agg, zero_agg)
    hg, lsm = _p3(agg3, h3p, dinv, b3.reshape(1, D_H),
                  batch.reshape(NRB, 1, RB), Wl, bl.reshape(1, N_CLS))
    return (hg, lsm)


# speedup vs baseline: 1.0574x; 1.0574x over previous
import functools

import jax
import jax.numpy as jnp
from jax import lax
from jax.experimental import pallas as pl
from jax.experimental.pallas import tpu as pltpu
from jax.experimental.pallas import tpu_sc as plsc

N = 10000
E = 160000
D_IN = 256
D_H = 512
G = 64
N_CLS = 6

NSC = 2
NSUB = 16
CH = 128
NPASS = 4
ROWS = 10112
RPS = ROWS // NSUB
KD = 40
E_DEG = 32 * KD * 128
KCA = 48
NCHA = 212
NSLOT = 4
E_AGG = NSUB * NCHA * KCA
RB = 1000
NRB = N // RB

_mesh = plsc.VectorSubcoreMesh(core_axis_name="c", subcore_axis_name="s",
                               num_cores=NSC, num_subcores=NSUB)


@functools.partial(
    pl.kernel,
    out_type=jax.ShapeDtypeStruct((NSC, ROWS, CH), jnp.float32),
    scratch_types=[
        pltpu.VMEM((128,), jnp.int32),
        pltpu.VMEM((128, CH), jnp.float32),
        pltpu.VMEM_SHARED((ROWS, CH), jnp.float32),
    ],
    mesh=_mesh,
)
def _deg_sc(dst_hbm, ones_hbm, zero_hbm, out_hbm, idx_v, ones_v, acc_sh):
    c = lax.axis_index("c")
    s = lax.axis_index("s")
    w = s * NSC + c
    pltpu.sync_copy(ones_hbm, ones_v)
    pltpu.sync_copy(zero_hbm, acc_sh.at[pl.ds(s * RPS, RPS)])
    plsc.subcore_barrier()

    def body(j, carry):
        pltpu.sync_copy(dst_hbm.at[w, j], idx_v)
        pltpu.sync_copy(ones_v, acc_sh.at[idx_v], add=True)
        return carry

    lax.fori_loop(0, KD, body, 0)
    plsc.subcore_barrier()
    pltpu.sync_copy(acc_sh.at[pl.ds(s * RPS, RPS)],
                    out_hbm.at[c, pl.ds(s * RPS, RPS)])


@functools.partial(
    pl.kernel,
    out_type=jax.ShapeDtypeStruct((NPASS, ROWS, CH), jnp.float32),
    scratch_types=(
        [pltpu.VMEM((NCHA * KCA,), jnp.int32),
         pltpu.VMEM((NCHA * KCA,), jnp.int32)]
        + [pltpu.VMEM((KCA, CH), jnp.float32) for _ in range(NSLOT)]
        + [pltpu.VMEM_SHARED((ROWS, CH), jnp.float32)]
        + [pltpu.SemaphoreType.DMA for _ in range(2 * NSLOT)]
    ),
    mesh=_mesh,
)
def _agg_sc(table_hbm, src_hbm, dst_hbm, zero_hbm, out_hbm,
            srcv, dstv, *rest):
    sts = rest[:NSLOT]
    acc_sh = rest[NSLOT]
    gsem = rest[NSLOT + 1:2 * NSLOT + 1]
    ssem = rest[2 * NSLOT + 1:]
    c = lax.axis_index("c")
    s = lax.axis_index("s")
    pltpu.sync_copy(dst_hbm.at[s], dstv)

    def gather(j, b):
        pltpu.async_copy(table_hbm.at[srcv.at[pl.ds(j * KCA, KCA)]],
                         sts[b], gsem[b])

    def gwait(b):
        pltpu.make_async_copy(table_hbm.at[srcv.at[pl.ds(0, KCA)]],
                              sts[b], gsem[b]).wait()

    def scatter(j, b):
        pltpu.async_copy(sts[b], acc_sh.at[dstv.at[pl.ds(j * KCA, KCA)]],
                         ssem[b], add=True)

    def swait(b):
        pltpu.make_async_copy(sts[b], acc_sh.at[dstv.at[pl.ds(0, KCA)]],
                              ssem[b]).wait()

    for k in range(2):
        p = c + 2 * k
        pltpu.sync_copy(src_hbm.at[p, s], srcv)
        pltpu.sync_copy(zero_hbm, acc_sh.at[pl.ds(s * RPS, RPS)])
        plsc.subcore_barrier()

        gather(0, 0)
        gather(1, 1)
        gather(2, 2)
        gwait(0)
        scatter(0, 0)
        gather(3, 3)
        gwait(1)
        scatter(1, 1)

        def body(tt, cc):
            for v in range(NSLOT):
                j = 2 + NSLOT * tt + v
                swait(v)
                gather(j + 2, v)
                gwait((2 + v) % NSLOT)
                scatter(j, (2 + v) % NSLOT)
            return cc

        lax.fori_loop(0, (NCHA - 4) // NSLOT, body, 0)
        gwait((NCHA - 2) % NSLOT)
        scatter(NCHA - 2, (NCHA - 2) % NSLOT)
        gwait((NCHA - 1) % NSLOT)
        scatter(NCHA - 1, (NCHA - 1) % NSLOT)
        for b in range(NSLOT):
            swait(b)
        plsc.subcore_barrier()
        pltpu.sync_copy(acc_sh.at[pl.ds(s * RPS, RPS)],
                        out_hbm.at[p, pl.ds(s * RPS, RPS)])
        if k == 0:
            plsc.subcore_barrier()


def _p0_body(x_ref, w_ref, deg_ref, hp_ref, dinv_ref):
    deg = deg_ref[0, :, 0:1] + deg_ref[1, :, 0:1] + 1.0
    dinv = lax.rsqrt(deg)
    h = jnp.dot(x_ref[...], w_ref[...], preferred_element_type=jnp.float32)
    hp = h * dinv
    dinv_ref[...] = dinv
    for ci in range(NPASS):
        hp_ref[ci] = hp[:, ci * CH:(ci + 1) * CH]


def _p12_body(agg_ref, hp_ref, dinv_ref, b_ref, w_ref, out_ref):
    dinv = dinv_ref[...]
    parts = []
    for ci in range(NPASS):
        t = dinv * (agg_ref[ci] + hp_ref[ci]) + b_ref[0:1, ci * CH:(ci + 1) * CH]
        parts.append(jnp.maximum(t, 0.0))
    t_full = jnp.concatenate(parts, axis=1)
    h = jnp.dot(t_full, w_ref[...], preferred_element_type=jnp.float32)
    hp = h * dinv
    for ci in range(NPASS):
        out_ref[ci] = hp[:, ci * CH:(ci + 1) * CH]


def _p3_body(agg_ref, hp_ref, dinv_ref, b_ref, batch_ref, wl_ref, bl_ref,
             hg_ref, lsm_ref, acc_ref, cnt_ref):
    i = pl.program_id(0)
    dinv = dinv_ref[...]
    parts = []
    for ci in range(NPASS):
        parts.append(dinv * (agg_ref[ci] + hp_ref[ci])
                     + b_ref[0:1, ci * CH:(ci + 1) * CH])
    h = jnp.concatenate(parts, axis=1)
    rows = lax.broadcasted_iota(jnp.int32, (G, RB), 0)
    oh = jnp.where(rows == batch_ref[0], 1.0, 0.0)
    pooled = jnp.dot(oh, h, preferred_element_type=jnp.float32)
    cnt = jnp.sum(oh, axis=1, keepdims=True)

    @pl.when(i == 0)
    def _init():
        acc_ref[...] = jnp.zeros((G, D_H), jnp.float32)
        cnt_ref[...] = jnp.zeros((G, 128), jnp.float32)

    acc_ref[...] += pooled
    cnt_ref[...] += jnp.broadcast_to(cnt, (G, 128))

    @pl.when(i == NRB - 1)
    def _fin():
        counts = cnt_ref[:, 0:1]
        hg = acc_ref[...] / jnp.maximum(counts, 1.0)
        hg_ref[...] = hg
        logits = jnp.dot(hg, wl_ref[...],
                         preferred_element_type=jnp.float32) + bl_ref[...]
        m = jnp.max(logits, axis=1, keepdims=True)
        ex = jnp.exp(logits - m)
        lsm_ref[...] = (logits - m) - jnp.log(jnp.sum(ex, axis=1, keepdims=True))


_f32 = jnp.float32

_p0 = pl.pallas_call(
    _p0_body,
    grid=(NRB,),
    in_specs=[
        pl.BlockSpec((RB, D_IN), lambda i: (i, 0)),
        pl.BlockSpec((D_IN, D_H), lambda i: (0, 0)),
        pl.BlockSpec((NSC, RB, CH), lambda i: (0, i, 0)),
    ],
    out_specs=[
        pl.BlockSpec((NPASS, RB, CH), lambda i: (0, i, 0)),
        pl.BlockSpec((RB, 1), lambda i: (i, 0)),
    ],
    out_shape=[
        jax.ShapeDtypeStruct((NPASS, N, CH), _f32),
        jax.ShapeDtypeStruct((N, 1), _f32),
    ],
)

_p12 = pl.pallas_call(
    _p12_body,
    grid=(NRB,),
    in_specs=[
        pl.BlockSpec((NPASS, RB, CH), lambda i: (0, i, 0)),
        pl.BlockSpec((NPASS, RB, CH), lambda i: (0, i, 0)),
        pl.BlockSpec((RB, 1), lambda i: (i, 0)),
        pl.BlockSpec((1, D_H), lambda i: (0, 0)),
        pl.BlockSpec((D_H, D_H), lambda i: (0, 0)),
    ],
    out_specs=[pl.BlockSpec((NPASS, RB, CH), lambda i: (0, i, 0))],
    out_shape=[jax.ShapeDtypeStruct((NPASS, N, CH), _f32)],
)

_p3 = pl.pallas_call(
    _p3_body,
    grid=(NRB,),
    in_specs=[
        pl.BlockSpec((NPASS, RB, CH), lambda i: (0, i, 0)),
        pl.BlockSpec((NPASS, RB, CH), lambda i: (0, i, 0)),
        pl.BlockSpec((RB, 1), lambda i: (i, 0)),
        pl.BlockSpec((1, D_H), lambda i: (0, 0)),
        pl.BlockSpec((1, 1, RB), lambda i: (i, 0, 0)),
        pl.BlockSpec((D_H, N_CLS), lambda i: (0, 0)),
        pl.BlockSpec((1, N_CLS), lambda i: (0, 0)),
    ],
    out_specs=[
        pl.BlockSpec((G, D_H), lambda i: (0, 0)),
        pl.BlockSpec((G, N_CLS), lambda i: (0, 0)),
    ],
    out_shape=[
        jax.ShapeDtypeStruct((G, D_H), _f32),
        jax.ShapeDtypeStruct((G, N_CLS), _f32),
    ],
    scratch_shapes=[
        pltpu.VMEM((G, D_H), _f32),
        pltpu.VMEM((G, 128), _f32),
    ],
)


def kernel(x, edge_index, batch, W1, b1, W2, b2, W3, b3, Wl, bl):
    src0 = edge_index[0]
    dst0 = edge_index[1]

    pad_d = jnp.full((E_DEG - E,), N, jnp.int32)
    dst_deg = jnp.concatenate([dst0, pad_d]).reshape(32, KD, 128)
    src_pad = jnp.concatenate([src0, jnp.zeros((E_AGG - E,), jnp.int32)])
    src_agg = (src_pad[None, :] + (jnp.arange(NPASS, dtype=jnp.int32) * N)[:, None]
               ).reshape(NPASS, NSUB, NCHA * KCA)
    dst_agg = jnp.concatenate(
        [dst0, jnp.full((E_AGG - E,), N, jnp.int32)]).reshape(NSUB, NCHA * KCA)

    ones128 = jnp.ones((128, CH), _f32)
    zero_agg = jnp.zeros((RPS, CH), _f32)

    degp = _deg_sc(dst_deg, ones128, zero_agg)

    h1p, dinv = _p0(x, W1, degp)
    agg1 = _agg_sc(h1p.reshape(NPASS * N, CH), src_agg, dst_agg, zero_agg)
    h2p, = _p12(agg1, h1p, dinv, b1.reshape(1, D_H), W2)
    agg2 = _agg_sc(h2p.reshape(NPASS * N, CH), src_agg, dst_agg, zero_agg)
    h3p, = _p12(agg2, h2p, dinv, b2.reshape(1, D_H), W3)
    agg3 = _agg_sc(h3p.reshape(NPASS * N, CH), src_agg, dst_agg, zero_agg)
    hg, lsm = _p3(agg3, h3p, dinv, b3.reshape(1, D_H),
                  batch.reshape(NRB, 1, RB), Wl, bl.reshape(1, N_CLS))
    return (hg, lsm)
